# baseline (speedup 1.0000x reference)
import functools

import jax
import jax.numpy as jnp
from jax import lax
from jax.experimental import pallas as pl
from jax.experimental.pallas import tpu as pltpu
from jax.experimental.pallas import tpu_sc as plsc

NW = 32
E_PER_W = 10000
CHUNKS = 125
C = 80
W = 64
GROUPS = C // 16
NBUF = 5


def _body(chem, dis, src, dst, out, src_v, dst_v,
          cb0, cb1, cb2, cb3, cb4, db0, db1, db2, db3, db4, out_v,
          sem0, sem1, sem2, sem3, sem4):
    cbufs = (cb0, cb1, cb2, cb3, cb4)
    dbufs = (db0, db1, db2, db3, db4)
    sems = (sem0, sem1, sem2, sem3, sem4)

    wid = lax.axis_index("s") * 2 + lax.axis_index("c")
    pltpu.sync_copy(src.at[wid], src_v)
    pltpu.sync_copy(dst.at[wid], dst_v)

    lanes = lax.iota(jnp.int32, 16)
    himask = jnp.full((16,), -65536, jnp.int32)

    H = C // 2

    def issue(c, b):
        for h in range(2):
            pltpu.async_copy(
                chem.at[src_v.at[pl.ds(c * C + h * H, H)]],
                cbufs[b].at[pl.ds(h * H, H)], sems[b])
            pltpu.async_copy(
                dis.at[dst_v.at[pl.ds(c * C + h * H, H)]],
                dbufs[b].at[pl.ds(h * H, H)], sems[b])

    def drain(b):
        for h in range(2):
            pltpu.make_async_copy(
                chem.at[src_v.at[pl.ds(0, H)]],
                cbufs[b].at[pl.ds(h * H, H)], sems[b]).wait()
            pltpu.make_async_copy(
                dis.at[dst_v.at[pl.ds(0, H)]],
                dbufs[b].at[pl.ds(h * H, H)], sems[b]).wait()

    def compute(c, b):
        def group(g, _):
            rows = g * 16 + lanes

            def wstep(w8, accs):
                acc0, acc1 = accs
                base = jnp.full((16,), 0, jnp.int32) + w8 * 8
                for j in range(8):
                    col = base + j
                    cw = plsc.load_gather(cbufs[b], [rows, col])
                    dw = plsc.load_gather(dbufs[b], [rows, col])
                    clo = plsc.bitcast(lax.shift_left(cw, 16), jnp.float32)
                    dlo = plsc.bitcast(lax.shift_left(dw, 16), jnp.float32)
                    chi = plsc.bitcast(lax.bitwise_and(cw, himask), jnp.float32)
                    dhi = plsc.bitcast(lax.bitwise_and(dw, himask), jnp.float32)
                    acc0 = acc0 + clo * dlo
                    acc1 = acc1 + chi * dhi
                return acc0, acc1

            zero = jnp.zeros((16,), jnp.float32)
            acc0, acc1 = lax.fori_loop(0, W // 8, wstep, (zero, zero))
            out_v[pl.ds(c * C + g * 16, 16)] = acc0 + acc1
            return 0

        lax.fori_loop(0, GROUPS, group, 0)

    for b in range(NBUF):
        issue(b, b)

    def step(k, _):
        for b in range(NBUF):
            c = k * NBUF + b
            drain(b)
            compute(c, b)

            @pl.when(c + NBUF < CHUNKS)
            def _():
                issue(c + NBUF, b)

        return 0

    lax.fori_loop(0, CHUNKS // NBUF, step, 0)
    pltpu.sync_copy(out_v, out.at[wid])


@jax.jit
def _run(chem_packed, dis_packed, src, dst):
    kfn = functools.partial(
        pl.kernel,
        mesh=plsc.VectorSubcoreMesh(core_axis_name="c", subcore_axis_name="s"),
        compiler_params=pltpu.CompilerParams(
            needs_layout_passes=False, use_tc_tiling_on_sc=False),
        out_type=jax.ShapeDtypeStruct((NW, E_PER_W), jnp.float32),
        scratch_types=[
            pltpu.VMEM((E_PER_W,), jnp.int32),
            pltpu.VMEM((E_PER_W,), jnp.int32),
        ] + [pltpu.VMEM((C, W), jnp.int32)] * (2 * NBUF) + [
            pltpu.VMEM((E_PER_W,), jnp.float32),
        ] + [pltpu.SemaphoreType.DMA] * NBUF,
    )(_body)
    return kfn(chem_packed, dis_packed, src, dst)


def _pack(table):
    r = lax.bitcast_convert_type(table, jnp.uint32)
    r = r + jnp.uint32(0x7FFF) + ((r >> 16) & jnp.uint32(1))
    w = (r[:, :64] >> 16) | (r[:, 64:] & jnp.uint32(0xFFFF0000))
    return lax.bitcast_convert_type(w, jnp.int32)


def kernel(chemical, disease, edge_label_index):
    idx = edge_label_index.astype(jnp.int32)
    src = idx[0].reshape(NW, E_PER_W)
    dst = idx[1].reshape(NW, E_PER_W)
    out = _run(_pack(chemical), _pack(disease), src, dst)
    return out.reshape(NW * E_PER_W)


# speedup vs baseline: 1.0447x; 1.0447x over previous
import functools

import jax
import jax.numpy as jnp
from jax import lax
from jax.experimental import pallas as pl
from jax.experimental.pallas import tpu as pltpu
from jax.experimental.pallas import tpu_sc as plsc

NW = 32
E_PER_W = 10000
CHUNKS = 125
C = 80
W = 64
GROUPS = C // 16
NBUF = 5


def _body(chem, dis, src, dst, out, src_v, dst_v,
          cb0, cb1, cb2, cb3, cb4, db0, db1, db2, db3, db4, out_v,
          sem0, sem1, sem2, sem3, sem4):
    cbufs = (cb0, cb1, cb2, cb3, cb4)
    dbufs = (db0, db1, db2, db3, db4)
    sems = (sem0, sem1, sem2, sem3, sem4)

    wid = lax.axis_index("s") * 2 + lax.axis_index("c")
    pltpu.sync_copy(src.at[wid], src_v)
    pltpu.sync_copy(dst.at[wid], dst_v)

    lanes = lax.iota(jnp.int32, 16)
    himask = jnp.full((16,), -65536, jnp.int32)

    H = C // 2

    def issue(c, b):
        for h in range(2):
            pltpu.async_copy(
                chem.at[src_v.at[pl.ds(c * C + h * H, H)]],
                cbufs[b].at[pl.ds(h * H, H)], sems[b])
            pltpu.async_copy(
                dis.at[dst_v.at[pl.ds(c * C + h * H, H)]],
                dbufs[b].at[pl.ds(h * H, H)], sems[b])

    def drain(b):
        for h in range(2):
            pltpu.make_async_copy(
                chem.at[src_v.at[pl.ds(0, H)]],
                cbufs[b].at[pl.ds(h * H, H)], sems[b]).wait()
            pltpu.make_async_copy(
                dis.at[dst_v.at[pl.ds(0, H)]],
                dbufs[b].at[pl.ds(h * H, H)], sems[b]).wait()

    def compute(c, b):
        def group(g, _):
            rows = g * 16 + lanes

            def wstep(w8, accs):
                acc0, acc1 = accs
                base = jnp.full((16,), 0, jnp.int32) + w8 * 8
                for j in range(8):
                    col = base + j
                    cw = plsc.load_gather(cbufs[b], [rows, col])
                    dw = plsc.load_gather(dbufs[b], [rows, col])
                    clo = plsc.bitcast(lax.shift_left(cw, 16), jnp.float32)
                    dlo = plsc.bitcast(lax.shift_left(dw, 16), jnp.float32)
                    chi = plsc.bitcast(lax.bitwise_and(cw, himask), jnp.float32)
                    dhi = plsc.bitcast(lax.bitwise_and(dw, himask), jnp.float32)
                    acc0 = acc0 + clo * dlo
                    acc1 = acc1 + chi * dhi
                return acc0, acc1

            zero = jnp.zeros((16,), jnp.float32)
            acc0, acc1 = lax.fori_loop(0, W // 8, wstep, (zero, zero))
            out_v[pl.ds(c * C + g * 16, 16)] = acc0 + acc1
            return 0

        lax.fori_loop(0, GROUPS, group, 0)

    for b in range(NBUF):
        issue(b, b)

    def step(k, _):
        for b in range(NBUF):
            c = k * NBUF + b
            drain(b)
            compute(c, b)

            @pl.when(c + NBUF < CHUNKS)
            def _():
                issue(c + NBUF, b)

        return 0

    lax.fori_loop(0, CHUNKS // NBUF, step, 0)
    pltpu.sync_copy(out_v, out.at[wid])


@jax.jit
def _run(chem_packed, dis_packed, src, dst):
    kfn = functools.partial(
        pl.kernel,
        mesh=plsc.VectorSubcoreMesh(core_axis_name="c", subcore_axis_name="s"),
        compiler_params=pltpu.CompilerParams(
            needs_layout_passes=False, use_tc_tiling_on_sc=False),
        out_type=jax.ShapeDtypeStruct((NW, E_PER_W), jnp.float32),
        scratch_types=[
            pltpu.VMEM((E_PER_W,), jnp.int32),
            pltpu.VMEM((E_PER_W,), jnp.int32),
        ] + [pltpu.VMEM((C, W), jnp.int32)] * (2 * NBUF) + [
            pltpu.VMEM((E_PER_W,), jnp.float32),
        ] + [pltpu.SemaphoreType.DMA] * NBUF,
    )(_body)
    return kfn(chem_packed, dis_packed, src, dst)


def _pack_body(x_ref, o_ref):
    r = lax.bitcast_convert_type(x_ref[...], jnp.uint32)
    r = r + jnp.uint32(0x7FFF) + ((r >> 16) & jnp.uint32(1))
    w = (r[:, :64] >> 16) | (r[:, 64:] & jnp.uint32(0xFFFF0000))
    o_ref[...] = lax.bitcast_convert_type(w, jnp.int32)


def _pack(table):
    n = table.shape[0]
    blk = 2000
    return pl.pallas_call(
        _pack_body,
        grid=(n // blk,),
        in_specs=[pl.BlockSpec((blk, 128), lambda i: (i, 0))],
        out_specs=pl.BlockSpec((blk, 64), lambda i: (i, 0)),
        out_shape=jax.ShapeDtypeStruct((n, 64), jnp.int32),
    )(table)


def kernel(chemical, disease, edge_label_index):
    idx = edge_label_index.astype(jnp.int32)
    src = idx[0].reshape(NW, E_PER_W)
    dst = idx[1].reshape(NW, E_PER_W)
    out = _run(_pack(chemical), _pack(disease), src, dst)
    return out.reshape(NW * E_PER_W)


# speedup vs baseline: 1.0691x; 1.0234x over previous
import functools

import jax
import jax.numpy as jnp
from jax import lax
from jax.experimental import pallas as pl
from jax.experimental.pallas import tpu as pltpu
from jax.experimental.pallas import tpu_sc as plsc

NW = 32
E_PER_W = 10000
CHUNKS = 125
C = 80
W = 64
GROUPS = C // 16
NBUF = 5


def _body(chem, dis, src, dst, out, src_v, dst_v,
          cb0, cb1, cb2, cb3, cb4, db0, db1, db2, db3, db4, out_v,
          sem0, sem1, sem2, sem3, sem4):
    cbufs = (cb0, cb1, cb2, cb3, cb4)
    dbufs = (db0, db1, db2, db3, db4)
    sems = (sem0, sem1, sem2, sem3, sem4)

    wid = lax.axis_index("s") * 2 + lax.axis_index("c")
    pltpu.sync_copy(src.at[wid], src_v)
    pltpu.sync_copy(dst.at[wid], dst_v)

    lanes = lax.iota(jnp.int32, 16)
    himask = jnp.full((16,), -65536, jnp.int32)

    H = C // 2

    def issue(c, b):
        for h in range(2):
            pltpu.async_copy(
                chem.at[src_v.at[pl.ds(c * C + h * H, H)]],
                cbufs[b].at[pl.ds(h * H, H)], sems[b])
            pltpu.async_copy(
                dis.at[dst_v.at[pl.ds(c * C + h * H, H)]],
                dbufs[b].at[pl.ds(h * H, H)], sems[b])

    def drain(b):
        for h in range(2):
            pltpu.make_async_copy(
                chem.at[src_v.at[pl.ds(0, H)]],
                cbufs[b].at[pl.ds(h * H, H)], sems[b]).wait()
            pltpu.make_async_copy(
                dis.at[dst_v.at[pl.ds(0, H)]],
                dbufs[b].at[pl.ds(h * H, H)], sems[b]).wait()

    def compute(c, b):
        def group(g, _):
            rows = g * 16 + lanes

            def wstep(w8, accs):
                acc0, acc1 = accs
                base = jnp.full((16,), 0, jnp.int32) + w8 * 8
                for j in range(8):
                    col = base + j
                    cw = plsc.load_gather(cbufs[b], [rows, col])
                    dw = plsc.load_gather(dbufs[b], [rows, col])
                    clo = plsc.bitcast(lax.shift_left(cw, 16), jnp.float32)
                    dlo = plsc.bitcast(lax.shift_left(dw, 16), jnp.float32)
                    chi = plsc.bitcast(lax.bitwise_and(cw, himask), jnp.float32)
                    dhi = plsc.bitcast(lax.bitwise_and(dw, himask), jnp.float32)
                    acc0 = acc0 + clo * dlo
                    acc1 = acc1 + chi * dhi
                return acc0, acc1

            zero = jnp.zeros((16,), jnp.float32)
            acc0, acc1 = lax.fori_loop(0, W // 8, wstep, (zero, zero))
            out_v[pl.ds(c * C + g * 16, 16)] = acc0 + acc1
            return 0

        lax.fori_loop(0, GROUPS, group, 0)

    for b in range(NBUF):
        issue(b, b)

    def step(k, _):
        for b in range(NBUF):
            c = k * NBUF + b
            drain(b)
            compute(c, b)

            @pl.when(c + NBUF < CHUNKS)
            def _():
                issue(c + NBUF, b)

        return 0

    lax.fori_loop(0, CHUNKS // NBUF, step, 0)
    pltpu.sync_copy(out_v, out.at[wid])


@jax.jit
def _run(chem_packed, dis_packed, src, dst):
    kfn = functools.partial(
        pl.kernel,
        mesh=plsc.VectorSubcoreMesh(core_axis_name="c", subcore_axis_name="s"),
        compiler_params=pltpu.CompilerParams(
            needs_layout_passes=False, use_tc_tiling_on_sc=False),
        out_type=jax.ShapeDtypeStruct((NW, E_PER_W), jnp.float32),
        scratch_types=[
            pltpu.VMEM((E_PER_W,), jnp.int32),
            pltpu.VMEM((E_PER_W,), jnp.int32),
        ] + [pltpu.VMEM((C, W), jnp.int32)] * (2 * NBUF) + [
            pltpu.VMEM((E_PER_W,), jnp.float32),
        ] + [pltpu.SemaphoreType.DMA] * NBUF,
    )(_body)
    return kfn(chem_packed, dis_packed, src, dst)


def _pack_body(x_ref, o_ref):
    r = lax.bitcast_convert_type(x_ref[...], jnp.uint32)
    r = r + jnp.uint32(0x7FFF) + ((r >> 16) & jnp.uint32(1))
    w = (r[:, :64] >> 16) | (r[:, 64:] & jnp.uint32(0xFFFF0000))
    o_ref[...] = lax.bitcast_convert_type(w, jnp.int32)


def _pack(table):
    n = table.shape[0]
    blk = 5000
    return pl.pallas_call(
        _pack_body,
        grid=(n // blk,),
        in_specs=[pl.BlockSpec((blk, 128), lambda i: (i, 0))],
        out_specs=pl.BlockSpec((blk, 64), lambda i: (i, 0)),
        out_shape=jax.ShapeDtypeStruct((n, 64), jnp.int32),
    )(table)


def kernel(chemical, disease, edge_label_index):
    idx = edge_label_index.astype(jnp.int32)
    src = idx[0].reshape(NW, E_PER_W)
    dst = idx[1].reshape(NW, E_PER_W)
    out = _run(_pack(chemical), _pack(disease), src, dst)
    return out.reshape(NW * E_PER_W)


# speedup vs baseline: 1.0747x; 1.0053x over previous
import functools

import jax
import jax.numpy as jnp
from jax import lax
from jax.experimental import pallas as pl
from jax.experimental.pallas import tpu as pltpu
from jax.experimental.pallas import tpu_sc as plsc

NW = 32
E_PER_W = 10000
CHUNKS = 125
C = 80
W = 64
GROUPS = C // 16
NBUF = 5


def _body(chem, dis, src, dst, out, src_v, dst_v,
          cb0, cb1, cb2, cb3, cb4, db0, db1, db2, db3, db4, out_v,
          sem0, sem1, sem2, sem3, sem4):
    cbufs = (cb0, cb1, cb2, cb3, cb4)
    dbufs = (db0, db1, db2, db3, db4)
    sems = (sem0, sem1, sem2, sem3, sem4)

    wid = lax.axis_index("s") * 2 + lax.axis_index("c")
    pltpu.sync_copy(src.at[wid], src_v)
    pltpu.sync_copy(dst.at[wid], dst_v)

    lanes = lax.iota(jnp.int32, 16)
    himask = jnp.full((16,), -65536, jnp.int32)

    H = C // 2

    def issue(c, b):
        for h in range(2):
            pltpu.async_copy(
                chem.at[src_v.at[pl.ds(c * C + h * H, H)]],
                cbufs[b].at[pl.ds(h * H, H)], sems[b])
            pltpu.async_copy(
                dis.at[dst_v.at[pl.ds(c * C + h * H, H)]],
                dbufs[b].at[pl.ds(h * H, H)], sems[b])

    def drain(b):
        for h in range(2):
            pltpu.make_async_copy(
                chem.at[src_v.at[pl.ds(0, H)]],
                cbufs[b].at[pl.ds(h * H, H)], sems[b]).wait()
            pltpu.make_async_copy(
                dis.at[dst_v.at[pl.ds(0, H)]],
                dbufs[b].at[pl.ds(h * H, H)], sems[b]).wait()

    def compute(c, b):
        def group(g, _):
            rows = g * 16 + lanes

            def wstep(w8, accs):
                acc0, acc1 = accs
                base = jnp.full((16,), 0, jnp.int32) + w8 * 8
                for j in range(8):
                    col = base + j
                    cw = plsc.load_gather(cbufs[b], [rows, col])
                    dw = plsc.load_gather(dbufs[b], [rows, col])
                    clo = plsc.bitcast(lax.shift_left(cw, 16), jnp.float32)
                    dlo = plsc.bitcast(lax.shift_left(dw, 16), jnp.float32)
                    chi = plsc.bitcast(lax.bitwise_and(cw, himask), jnp.float32)
                    dhi = plsc.bitcast(lax.bitwise_and(dw, himask), jnp.float32)
                    acc0 = acc0 + clo * dlo
                    acc1 = acc1 + chi * dhi
                return acc0, acc1

            zero = jnp.zeros((16,), jnp.float32)
            acc0, acc1 = lax.fori_loop(0, W // 8, wstep, (zero, zero))
            out_v[pl.ds(c * C + g * 16, 16)] = acc0 + acc1
            return 0

        lax.fori_loop(0, GROUPS, group, 0)

    for b in range(NBUF):
        issue(b, b)

    def step(k, _):
        for b in range(NBUF):
            c = k * NBUF + b
            drain(b)
            compute(c, b)

            @pl.when(c + NBUF < CHUNKS)
            def _():
                issue(c + NBUF, b)

        return 0

    lax.fori_loop(0, CHUNKS // NBUF, step, 0)
    pltpu.sync_copy(out_v, out.at[wid])


@jax.jit
def _run(chem_packed, dis_packed, src, dst):
    kfn = functools.partial(
        pl.kernel,
        mesh=plsc.VectorSubcoreMesh(core_axis_name="c", subcore_axis_name="s"),
        compiler_params=pltpu.CompilerParams(
            needs_layout_passes=False, use_tc_tiling_on_sc=False),
        out_type=jax.ShapeDtypeStruct((NW, E_PER_W), jnp.float32),
        scratch_types=[
            pltpu.VMEM((E_PER_W,), jnp.int32),
            pltpu.VMEM((E_PER_W,), jnp.int32),
        ] + [pltpu.VMEM((C, W), jnp.int32)] * (2 * NBUF) + [
            pltpu.VMEM((E_PER_W,), jnp.float32),
        ] + [pltpu.SemaphoreType.DMA] * NBUF,
    )(_body)
    return kfn(chem_packed, dis_packed, src, dst)


def _pack_body(x_ref, o_ref):
    r = lax.bitcast_convert_type(x_ref[...], jnp.uint32)
    r = r + jnp.uint32(0x7FFF) + ((r >> 16) & jnp.uint32(1))
    w = (r[:, :64] >> 16) | (r[:, 64:] & jnp.uint32(0xFFFF0000))
    o_ref[...] = lax.bitcast_convert_type(w, jnp.int32)


def _pack(table):
    n = table.shape[0]
    blk = 10000
    return pl.pallas_call(
        _pack_body,
        grid=(n // blk,),
        in_specs=[pl.BlockSpec((blk, 128), lambda i: (i, 0))],
        out_specs=pl.BlockSpec((blk, 64), lambda i: (i, 0)),
        out_shape=jax.ShapeDtypeStruct((n, 64), jnp.int32),
    )(table)


def kernel(chemical, disease, edge_label_index):
    idx = edge_label_index.astype(jnp.int32)
    src = idx[0].reshape(NW, E_PER_W)
    dst = idx[1].reshape(NW, E_PER_W)
    out = _run(_pack(chemical), _pack(disease), src, dst)
    return out.reshape(NW * E_PER_W)
